# baseline (speedup 1.0000x reference)
import dataclasses
import functools

import jax
import jax.numpy as jnp
from jax import lax
from jax.experimental import pallas as pl
from jax.experimental.pallas import tpu as pltpu
from jax.experimental.pallas import tpu_sc as plsc

N = 10000
E = 320000
EP = 100000
D = 128
L = 3

NC = 2
NS = 16
NW = NC * NS
NPAD = 10240

HNODE = NPAD // NC
ACCR = HNODE + 16
HROW = HNODE // NS
ECW = 128
FCW = 80

EBW = 80
DECH = 125
LCH = 80
LSZ = LCH * ECW

PCH = 49
PCW = 128
P2 = NW * PCH * PCW

_f32 = jnp.float32
_i32 = jnp.int32


def _pre_body(src_hbm, dst_hbm, dop_hbm, dip_hbm, lsrc_hbm, ldst_hbm,
              cnt_hbm, sidx_v, didx_v, hs_v, hd_v, ls0, ld0, ls1, ld1, cnt_v):
    cid = lax.axis_index("c")
    sid = lax.axis_index("s")
    wid = cid * NS + sid
    pltpu.sync_copy(src_hbm.at[wid], sidx_v)
    pltpu.sync_copy(dst_hbm.at[wid], didx_v)

    @pl.loop(0, NPAD // 16)
    def _zero(i):
        z = jnp.zeros((16,), _f32)
        hs_v[pl.ds(i * 16, 16)] = z
        hd_v[pl.ds(i * 16, 16)] = z

    ones = jnp.ones((16,), _f32)

    @pl.loop(0, DECH, init_carry=(0, 0))
    def offs(j, carry):
        o0, o1 = carry
        for k in range(EBW // 16):
            sl = pl.ds(k * 16, 16)
            s = sidx_v[j, sl]
            d = didx_v[j, sl]
            plsc.addupdate_scatter(hs_v, [s], ones)
            plsc.addupdate_scatter(hd_v, [d], ones)
            m0 = d < HNODE
            c0 = jnp.sum(m0.astype(_i32))
            plsc.store_compressed(ls0.at[pl.ds(o0, 16)], s, mask=m0)
            plsc.store_compressed(ld0.at[pl.ds(o0, 16)], d, mask=m0)
            m1 = jnp.logical_not(m0)
            plsc.store_compressed(ls1.at[pl.ds(o1, 16)], s, mask=m1)
            plsc.store_compressed(ld1.at[pl.ds(o1, 16)], d - HNODE, mask=m1)
            o0 = o0 + c0
            o1 = o1 + (16 - c0)
        return o0, o1

    o0, o1 = offs
    iota = lax.iota(_i32, 16)
    dummy_d = jnp.full((16,), HNODE, _i32)
    dummy_s = jnp.zeros((16,), _i32)
    for k in range(ECW // 16):
        plsc.store_scatter(ld0, [o0 + iota + k * 16], dummy_d)
        plsc.store_scatter(ls0, [o0 + iota + k * 16], dummy_s)
        plsc.store_scatter(ld1, [o1 + iota + k * 16], dummy_d)
        plsc.store_scatter(ls1, [o1 + iota + k * 16], dummy_s)
    cnt_v[pl.ds(0, 16)] = jnp.where(iota == 0, o0,
                                    jnp.where(iota == 1, o1, 0))

    pltpu.sync_copy(hs_v, dop_hbm.at[wid])
    pltpu.sync_copy(hd_v, dip_hbm.at[wid])
    pltpu.sync_copy(ls0, lsrc_hbm.at[0, wid])
    pltpu.sync_copy(ld0, ldst_hbm.at[0, wid])
    pltpu.sync_copy(ls1, lsrc_hbm.at[1, wid])
    pltpu.sync_copy(ld1, ldst_hbm.at[1, wid])
    pltpu.sync_copy(cnt_v, cnt_hbm.at[wid])


def _conv_body(g_hbm, lsrc_hbm, ldst_hbm, cnt_hbm, zer_hbm, out_hbm,
               sidx_v, didx_v, rows_a, rows_b, rows_c, rows_d, cnt_s, acc_sh,
               sem_a, sem_b, sem_c, sem_d, ssem):
    cid = lax.axis_index("c")
    sid = lax.axis_index("s")
    rows = (rows_a, rows_b, rows_c, rows_d)
    sems = (sem_a, sem_b, sem_c, sem_d)
    pltpu.sync_copy(zer_hbm, acc_sh.at[pl.ds(sid * HROW, HROW)])
    plsc.subcore_barrier()

    for li in range(2):
        w = sid * 2 + li
        pltpu.sync_copy(lsrc_hbm.at[cid, w], sidx_v)
        pltpu.sync_copy(ldst_hbm.at[cid, w], didx_v)
        pltpu.sync_copy(cnt_hbm.at[w], cnt_s)
        cv = cnt_s[pl.ds(0, 16)]
        cnt = jnp.where(cid == 0, cv[0], cv[1])
        nch = lax.div(cnt + (ECW - 1), ECW)

        for m in range(3):
            @pl.when(m < nch)
            def _():
                pltpu.async_copy(g_hbm.at[sidx_v.at[m]], rows[m], sems[m])

        @pl.loop(0, lax.div(nch + 3, 4))
        def _edges(jq):
            j = jq * 4
            for m in range(4):
                jj = j + m
                @pl.when(jj < nch)
                def _():
                    pltpu.make_async_copy(
                        g_hbm.at[sidx_v.at[jj]], rows[m], sems[m]).wait()
                    @pl.when(jj >= 1)
                    def _():
                        pltpu.make_async_copy(
                            rows[0], acc_sh.at[didx_v.at[0]], ssem).wait()
                    @pl.when(jj + 3 < nch)
                    def _():
                        pltpu.async_copy(g_hbm.at[sidx_v.at[jj + 3]],
                                         rows[(m + 3) % 4], sems[(m + 3) % 4])
                    pltpu.async_copy(rows[m], acc_sh.at[didx_v.at[jj]],
                                     ssem, add=True)

        @pl.when(nch > 0)
        def _():
            pltpu.make_async_copy(rows[0], acc_sh.at[didx_v.at[0]],
                                  ssem).wait()

    plsc.subcore_barrier()
    pltpu.sync_copy(acc_sh.at[pl.ds(sid * HROW, HROW)],
                    out_hbm.at[cid, pl.ds(sid * HROW, HROW)])


def _pair_body(rep_hbm, ia_hbm, ib_hbm, z_hbm,
               ia_v, ib_v, ra0, rb0, ra1, rb1, sem0, sem1):
    cid = lax.axis_index("c")
    sid = lax.axis_index("s")
    wid = cid * NS + sid
    pltpu.sync_copy(ia_hbm.at[wid], ia_v)
    pltpu.sync_copy(ib_hbm.at[wid], ib_v)
    base = wid * (PCH * PCW)

    def work(j, ra, rb, sem, ran, rbn, semn):
        pltpu.make_async_copy(rep_hbm.at[ia_v.at[j]], ra, sem).wait()
        pltpu.make_async_copy(rep_hbm.at[ib_v.at[j]], rb, sem).wait()

        @pl.when(j + 1 < PCH)
        def _():
            pltpu.async_copy(rep_hbm.at[ia_v.at[j + 1]], ran, semn)
            pltpu.async_copy(rep_hbm.at[ib_v.at[j + 1]], rbn, semn)

        @pl.loop(0, PCW)
        def _row(r):
            for k in range(D // 16):
                sl = pl.ds(k * 16, 16)
                ra[r, sl] = ra[r, sl] * rb[r, sl]

        pltpu.sync_copy(ra, z_hbm.at[pl.ds(base + j * PCW, PCW)])

    pltpu.async_copy(rep_hbm.at[ia_v.at[0]], ra0, sem0)
    pltpu.async_copy(rep_hbm.at[ib_v.at[0]], rb0, sem0)

    @pl.loop(0, PCH // 2)
    def _chunk(jh):
        j = jh * 2
        work(j, ra0, rb0, sem0, ra1, rb1, sem1)
        work(j + 1, ra1, rb1, sem1, ra0, rb0, sem0)

    work(PCH - 1, ra0, rb0, sem0, ra1, rb1, sem1)


@functools.cache
def _sc_kernels():
    mesh = plsc.VectorSubcoreMesh(
        core_axis_name="c", subcore_axis_name="s",
        num_cores=NC, num_subcores=NS,
    )
    cp = pltpu.CompilerParams()
    if "needs_layout_passes" in pltpu.CompilerParams.__dataclass_fields__:
        cp = dataclasses.replace(cp, needs_layout_passes=False)
    pre = pl.kernel(
        _pre_body,
        out_type=(
            jax.ShapeDtypeStruct((NW, NPAD), _f32),
            jax.ShapeDtypeStruct((NW, NPAD), _f32),
            jax.ShapeDtypeStruct((NC, NW, LSZ), _i32),
            jax.ShapeDtypeStruct((NC, NW, LSZ), _i32),
            jax.ShapeDtypeStruct((NW, 16), _i32),
        ),
        mesh=mesh,
        scratch_types=[
            pltpu.VMEM((DECH, EBW), _i32),
            pltpu.VMEM((DECH, EBW), _i32),
            pltpu.VMEM((NPAD,), _f32),
            pltpu.VMEM((NPAD,), _f32),
            pltpu.VMEM((LSZ,), _i32),
            pltpu.VMEM((LSZ,), _i32),
            pltpu.VMEM((LSZ,), _i32),
            pltpu.VMEM((LSZ,), _i32),
            pltpu.VMEM((16,), _i32),
        ],
        compiler_params=cp,
    )
    conv = pl.kernel(
        _conv_body,
        out_type=jax.ShapeDtypeStruct((NC, HNODE, D), _f32),
        mesh=mesh,
        scratch_types=[
            pltpu.VMEM((LCH, ECW), _i32),
            pltpu.VMEM((LCH, ECW), _i32),
            pltpu.VMEM((ECW, D), _f32),
            pltpu.VMEM((ECW, D), _f32),
            pltpu.VMEM((ECW, D), _f32),
            pltpu.VMEM((ECW, D), _f32),
            pltpu.VMEM((16,), _i32),
            pltpu.VMEM_SHARED((ACCR, D), _f32),
            pltpu.SemaphoreType.DMA,
            pltpu.SemaphoreType.DMA,
            pltpu.SemaphoreType.DMA,
            pltpu.SemaphoreType.DMA,
            pltpu.SemaphoreType.DMA,
        ],
        compiler_params=cp,
    )
    pair = pl.kernel(
        _pair_body,
        out_type=jax.ShapeDtypeStruct((P2, D), _f32),
        mesh=mesh,
        scratch_types=[
            pltpu.VMEM((PCH, PCW), _i32),
            pltpu.VMEM((PCH, PCW), _i32),
            pltpu.VMEM((PCW, D), _f32),
            pltpu.VMEM((PCW, D), _f32),
            pltpu.VMEM((PCW, D), _f32),
            pltpu.VMEM((PCW, D), _f32),
            pltpu.SemaphoreType.DMA,
            pltpu.SemaphoreType.DMA,
        ],
        compiler_params=cp,
    )
    return pre, conv, pair


_MLP_R = 1000


def _mlp_tc(x, dop_t, dip_t, w1, b1, w2, b2, gam):
    def body(x_ref, dop_ref, dip_ref, w1_ref, b1_ref, w2_ref, b2_ref, g_ref,
             rep_ref, gout_ref, ci_ref, cp_ref):
        dout = jnp.sum(dop_ref[...], axis=1, keepdims=True)
        din = jnp.sum(dip_ref[...], axis=1, keepdims=True)
        co = lax.rsqrt(jnp.maximum(dout, 1.0))
        ci = lax.rsqrt(jnp.maximum(din, 1.0))
        h = jnp.dot(x_ref[...], w1_ref[...], preferred_element_type=_f32)
        h = jnp.maximum(h + b1_ref[...], 0.0)
        h = jnp.dot(h, w2_ref[...], preferred_element_type=_f32) + b2_ref[...]
        rep_ref[...] = g_ref[0, 0] * h
        gout_ref[...] = co * h
        ci_ref[...] = ci
        cp_ref[...] = co * ci

    full = lambda s: pl.BlockSpec(s, lambda i: (0, 0))
    return pl.pallas_call(
        body,
        grid=(N // _MLP_R,),
        in_specs=[
            pl.BlockSpec((_MLP_R, D), lambda i: (i, 0)),
            pl.BlockSpec((_MLP_R, NW), lambda i: (i, 0)),
            pl.BlockSpec((_MLP_R, NW), lambda i: (i, 0)),
            full((D, D)), full((1, D)), full((D, D)), full((1, D)),
            full((1, L + 1)),
        ],
        out_specs=(
            pl.BlockSpec((_MLP_R, D), lambda i: (i, 0)),
            pl.BlockSpec((_MLP_R, D), lambda i: (i, 0)),
            pl.BlockSpec((_MLP_R, 1), lambda i: (i, 0)),
            pl.BlockSpec((_MLP_R, 1), lambda i: (i, 0)),
        ),
        out_shape=(
            jax.ShapeDtypeStruct((N, D), _f32),
            jax.ShapeDtypeStruct((N, D), _f32),
            jax.ShapeDtypeStruct((N, 1), _f32),
            jax.ShapeDtypeStruct((N, 1), _f32),
        ),
    )(x, dop_t, dip_t, w1, b1, w2, b2, gam)


def _scale_tc(part, rep, cin, cprod, gam, layer, last):
    def body(part_ref, rep_ref, ci_ref, cp_ref, g_ref, *outs):
        s = part_ref[...]
        outs[0][...] = rep_ref[...] + g_ref[0, layer + 1] * (ci_ref[...] * s)
        if not last:
            outs[1][...] = cp_ref[...] * s

    n_out = 1 if last else 2
    out = pl.pallas_call(
        body,
        grid=(N // _MLP_R,),
        in_specs=[
            pl.BlockSpec((_MLP_R, D), lambda i: (i, 0)),
            pl.BlockSpec((_MLP_R, D), lambda i: (i, 0)),
            pl.BlockSpec((_MLP_R, 1), lambda i: (i, 0)),
            pl.BlockSpec((_MLP_R, 1), lambda i: (i, 0)),
            pl.BlockSpec((1, L + 1), lambda i: (0, 0)),
        ],
        out_specs=(pl.BlockSpec((_MLP_R, D), lambda i: (i, 0)),) * n_out,
        out_shape=(jax.ShapeDtypeStruct((N, D), _f32),) * n_out,
    )(part, rep, cin, cprod, gam)
    return out if not last else (out[0], None)


_PRED_R = 2048


def _pred_tc(z, w1, b1, w2, b2, w3, b3):
    def body(z_ref, w1_ref, b1_ref, w2_ref, b2_ref, w3_ref, b3_ref, o_ref):
        a = jnp.dot(z_ref[...], w1_ref[...], preferred_element_type=_f32)
        a = jnp.maximum(a + b1_ref[...], 0.0)
        a = jnp.dot(a, w2_ref[...], preferred_element_type=_f32)
        a = jnp.maximum(a + b2_ref[...], 0.0)
        o_ref[...] = (jnp.dot(a, w3_ref[...], preferred_element_type=_f32)
                      + b3_ref[...])

    full = lambda s: pl.BlockSpec(s, lambda i: (0, 0))
    return pl.pallas_call(
        body,
        grid=(P2 // _PRED_R,),
        in_specs=[
            pl.BlockSpec((_PRED_R, D), lambda i: (i, 0)),
            full((D, D)), full((1, D)), full((D, D)), full((1, D)),
            full((D, 1)), full((1, 1)),
        ],
        out_specs=pl.BlockSpec((_PRED_R, 1), lambda i: (i, 0)),
        out_shape=jax.ShapeDtypeStruct((P2, 1), _f32),
    )(z, w1, b1, w2, b2, w3, b3)


def kernel(x, edge_index, pos_edge_index, neg_edge_index, lin1_W, lin1_b,
           lin2_W, lin2_b, gamma, p1_W, p1_b, p2_W, p2_b, p3_W, p3_b):
    gam = gamma.reshape(1, L + 1)
    _pre_sc, _conv_sc, _pair_sc = _sc_kernels()

    dop, dip, lsrc, ldst, cnt = _pre_sc(edge_index[0].reshape(NW, DECH, EBW),
                                        edge_index[1].reshape(NW, DECH, EBW))
    lsrc_r = lsrc.reshape(NC, NW, LCH, ECW)
    ldst_r = ldst.reshape(NC, NW, LCH, ECW)
    rep, g, cin, cprod = _mlp_tc(x, dop.T, dip.T, lin1_W,
                                 lin1_b.reshape(1, D), lin2_W,
                                 lin2_b.reshape(1, D), gam)

    zer = jnp.zeros((HROW, D), _f32)
    for l in range(L):
        part = _conv_sc(g, lsrc_r, ldst_r, cnt, zer).reshape(NPAD, D)[:N]
        rep, g = _scale_tc(part, rep, cin, cprod, gam, l, last=(l == L - 1))

    pad = jnp.zeros((P2 - 2 * EP,), _i32)
    ia = jnp.concatenate([pos_edge_index[0], neg_edge_index[0], pad])
    ib = jnp.concatenate([pos_edge_index[1], neg_edge_index[1], pad])
    z = _pair_sc(rep, ia.reshape(NW, PCH, PCW), ib.reshape(NW, PCH, PCW))

    scores = _pred_tc(z, p1_W, p1_b.reshape(1, D), p2_W, p2_b.reshape(1, D),
                      p3_W, p3_b.reshape(1, 1))
    return scores[:EP], scores[EP:2 * EP]


# speedup vs baseline: 1.0002x; 1.0002x over previous
import dataclasses
import functools

import jax
import jax.numpy as jnp
from jax import lax
from jax.experimental import pallas as pl
from jax.experimental.pallas import tpu as pltpu
from jax.experimental.pallas import tpu_sc as plsc

N = 10000
E = 320000
EP = 100000
D = 128
L = 3

NC = 2
NS = 16
NW = NC * NS
NPAD = 10240

HNODE = NPAD // NC
ACCR = HNODE + 16
HROW = HNODE // NS
ECW = 128
FCW = 80

EBW = 80
DECH = 125
LCH = 80
LSZ = LCH * ECW

PCH = 49
PCW = 128
P2 = NW * PCH * PCW

_f32 = jnp.float32
_i32 = jnp.int32


def _pre_body(src_hbm, dst_hbm, dop_hbm, dip_hbm, lsrc_hbm, ldst_hbm,
              cnt_hbm, sidx_v, didx_v, hs_v, hd_v, ls0, ld0, ls1, ld1, cnt_v):
    cid = lax.axis_index("c")
    sid = lax.axis_index("s")
    wid = cid * NS + sid
    pltpu.sync_copy(src_hbm.at[wid], sidx_v)
    pltpu.sync_copy(dst_hbm.at[wid], didx_v)

    @pl.loop(0, NPAD // 16)
    def _zero(i):
        z = jnp.zeros((16,), _f32)
        hs_v[pl.ds(i * 16, 16)] = z
        hd_v[pl.ds(i * 16, 16)] = z

    ones = jnp.ones((16,), _f32)

    @pl.loop(0, DECH, init_carry=(0, 0))
    def offs(j, carry):
        o0, o1 = carry
        for k in range(EBW // 16):
            sl = pl.ds(k * 16, 16)
            s = sidx_v[j, sl]
            d = didx_v[j, sl]
            plsc.addupdate_scatter(hs_v, [s], ones)
            plsc.addupdate_scatter(hd_v, [d], ones)
            m0 = d < HNODE
            c0 = jnp.sum(m0.astype(_i32))
            plsc.store_compressed(ls0.at[pl.ds(o0, 16)], s, mask=m0)
            plsc.store_compressed(ld0.at[pl.ds(o0, 16)], d, mask=m0)
            m1 = jnp.logical_not(m0)
            plsc.store_compressed(ls1.at[pl.ds(o1, 16)], s, mask=m1)
            plsc.store_compressed(ld1.at[pl.ds(o1, 16)], d - HNODE, mask=m1)
            o0 = o0 + c0
            o1 = o1 + (16 - c0)
        return o0, o1

    o0, o1 = offs
    iota = lax.iota(_i32, 16)
    dummy_d = jnp.full((16,), HNODE, _i32)
    dummy_s = jnp.zeros((16,), _i32)
    for k in range(ECW // 16):
        plsc.store_scatter(ld0, [o0 + iota + k * 16], dummy_d)
        plsc.store_scatter(ls0, [o0 + iota + k * 16], dummy_s)
        plsc.store_scatter(ld1, [o1 + iota + k * 16], dummy_d)
        plsc.store_scatter(ls1, [o1 + iota + k * 16], dummy_s)
    cnt_v[pl.ds(0, 16)] = jnp.where(iota == 0, o0,
                                    jnp.where(iota == 1, o1, 0))

    pltpu.sync_copy(hs_v, dop_hbm.at[wid])
    pltpu.sync_copy(hd_v, dip_hbm.at[wid])
    pltpu.sync_copy(ls0, lsrc_hbm.at[0, wid])
    pltpu.sync_copy(ld0, ldst_hbm.at[0, wid])
    pltpu.sync_copy(ls1, lsrc_hbm.at[1, wid])
    pltpu.sync_copy(ld1, ldst_hbm.at[1, wid])
    pltpu.sync_copy(cnt_v, cnt_hbm.at[wid])


def _conv_body(g_hbm, lsrc_hbm, ldst_hbm, cnt_hbm, zer_hbm, out_hbm,
               sidx_v, didx_v, rows_a, rows_b, rows_c, rows_d, cnt_s, acc_sh,
               sem_a, sem_b, sem_c, sem_d):
    cid = lax.axis_index("c")
    sid = lax.axis_index("s")
    rows = (rows_a, rows_b, rows_c, rows_d)
    sems = (sem_a, sem_b, sem_c, sem_d)
    pltpu.sync_copy(zer_hbm, acc_sh.at[pl.ds(sid * HROW, HROW)])
    plsc.subcore_barrier()

    for li in range(2):
        w = sid * 2 + li
        pltpu.sync_copy(lsrc_hbm.at[cid, w], sidx_v)
        pltpu.sync_copy(ldst_hbm.at[cid, w], didx_v)
        pltpu.sync_copy(cnt_hbm.at[w], cnt_s)
        cv = cnt_s[pl.ds(0, 16)]
        cnt = jnp.where(cid == 0, cv[0], cv[1])
        nch = lax.div(cnt + (ECW - 1), ECW)

        for m in range(3):
            @pl.when(m < nch)
            def _():
                pltpu.async_copy(g_hbm.at[sidx_v.at[m]], rows[m], sems[m])

        @pl.loop(0, lax.div(nch + 3, 4))
        def _edges(jq):
            j = jq * 4
            for m in range(4):
                jj = j + m
                @pl.when(jj < nch)
                def _():
                    pltpu.make_async_copy(
                        g_hbm.at[sidx_v.at[jj]], rows[m], sems[m]).wait()
                    @pl.when(jj + 3 < nch)
                    def _():
                        pltpu.async_copy(g_hbm.at[sidx_v.at[jj + 3]],
                                         rows[(m + 3) % 4], sems[(m + 3) % 4])
                    pltpu.sync_copy(rows[m], acc_sh.at[didx_v.at[jj]],
                                    add=True)

    plsc.subcore_barrier()
    pltpu.sync_copy(acc_sh.at[pl.ds(sid * HROW, HROW)],
                    out_hbm.at[cid, pl.ds(sid * HROW, HROW)])


def _pair_body(rep_hbm, ia_hbm, ib_hbm, z_hbm,
               ia_v, ib_v, ra0, rb0, ra1, rb1, sem0, sem1):
    cid = lax.axis_index("c")
    sid = lax.axis_index("s")
    wid = cid * NS + sid
    pltpu.sync_copy(ia_hbm.at[wid], ia_v)
    pltpu.sync_copy(ib_hbm.at[wid], ib_v)
    base = wid * (PCH * PCW)

    def work(j, ra, rb, sem, ran, rbn, semn):
        pltpu.make_async_copy(rep_hbm.at[ia_v.at[j]], ra, sem).wait()
        pltpu.make_async_copy(rep_hbm.at[ib_v.at[j]], rb, sem).wait()

        @pl.when(j + 1 < PCH)
        def _():
            pltpu.async_copy(rep_hbm.at[ia_v.at[j + 1]], ran, semn)
            pltpu.async_copy(rep_hbm.at[ib_v.at[j + 1]], rbn, semn)

        @pl.loop(0, PCW)
        def _row(r):
            for k in range(D // 16):
                sl = pl.ds(k * 16, 16)
                ra[r, sl] = ra[r, sl] * rb[r, sl]

        pltpu.sync_copy(ra, z_hbm.at[pl.ds(base + j * PCW, PCW)])

    pltpu.async_copy(rep_hbm.at[ia_v.at[0]], ra0, sem0)
    pltpu.async_copy(rep_hbm.at[ib_v.at[0]], rb0, sem0)

    @pl.loop(0, PCH // 2)
    def _chunk(jh):
        j = jh * 2
        work(j, ra0, rb0, sem0, ra1, rb1, sem1)
        work(j + 1, ra1, rb1, sem1, ra0, rb0, sem0)

    work(PCH - 1, ra0, rb0, sem0, ra1, rb1, sem1)


@functools.cache
def _sc_kernels():
    mesh = plsc.VectorSubcoreMesh(
        core_axis_name="c", subcore_axis_name="s",
        num_cores=NC, num_subcores=NS,
    )
    cp = pltpu.CompilerParams()
    if "needs_layout_passes" in pltpu.CompilerParams.__dataclass_fields__:
        cp = dataclasses.replace(cp, needs_layout_passes=False)
    pre = pl.kernel(
        _pre_body,
        out_type=(
            jax.ShapeDtypeStruct((NW, NPAD), _f32),
            jax.ShapeDtypeStruct((NW, NPAD), _f32),
            jax.ShapeDtypeStruct((NC, NW, LSZ), _i32),
            jax.ShapeDtypeStruct((NC, NW, LSZ), _i32),
            jax.ShapeDtypeStruct((NW, 16), _i32),
        ),
        mesh=mesh,
        scratch_types=[
            pltpu.VMEM((DECH, EBW), _i32),
            pltpu.VMEM((DECH, EBW), _i32),
            pltpu.VMEM((NPAD,), _f32),
            pltpu.VMEM((NPAD,), _f32),
            pltpu.VMEM((LSZ,), _i32),
            pltpu.VMEM((LSZ,), _i32),
            pltpu.VMEM((LSZ,), _i32),
            pltpu.VMEM((LSZ,), _i32),
            pltpu.VMEM((16,), _i32),
        ],
        compiler_params=cp,
    )
    conv = pl.kernel(
        _conv_body,
        out_type=jax.ShapeDtypeStruct((NC, HNODE, D), _f32),
        mesh=mesh,
        scratch_types=[
            pltpu.VMEM((LCH, ECW), _i32),
            pltpu.VMEM((LCH, ECW), _i32),
            pltpu.VMEM((ECW, D), _f32),
            pltpu.VMEM((ECW, D), _f32),
            pltpu.VMEM((ECW, D), _f32),
            pltpu.VMEM((ECW, D), _f32),
            pltpu.VMEM((16,), _i32),
            pltpu.VMEM_SHARED((ACCR, D), _f32),
            pltpu.SemaphoreType.DMA,
            pltpu.SemaphoreType.DMA,
            pltpu.SemaphoreType.DMA,
            pltpu.SemaphoreType.DMA,
        ],
        compiler_params=cp,
    )
    pair = pl.kernel(
        _pair_body,
        out_type=jax.ShapeDtypeStruct((P2, D), _f32),
        mesh=mesh,
        scratch_types=[
            pltpu.VMEM((PCH, PCW), _i32),
            pltpu.VMEM((PCH, PCW), _i32),
            pltpu.VMEM((PCW, D), _f32),
            pltpu.VMEM((PCW, D), _f32),
            pltpu.VMEM((PCW, D), _f32),
            pltpu.VMEM((PCW, D), _f32),
            pltpu.SemaphoreType.DMA,
            pltpu.SemaphoreType.DMA,
        ],
        compiler_params=cp,
    )
    return pre, conv, pair


_MLP_R = 1000


def _mlp_tc(x, dop_t, dip_t, w1, b1, w2, b2, gam):
    def body(x_ref, dop_ref, dip_ref, w1_ref, b1_ref, w2_ref, b2_ref, g_ref,
             rep_ref, gout_ref, ci_ref, cp_ref):
        dout = jnp.sum(dop_ref[...], axis=1, keepdims=True)
        din = jnp.sum(dip_ref[...], axis=1, keepdims=True)
        co = lax.rsqrt(jnp.maximum(dout, 1.0))
        ci = lax.rsqrt(jnp.maximum(din, 1.0))
        h = jnp.dot(x_ref[...], w1_ref[...], preferred_element_type=_f32)
        h = jnp.maximum(h + b1_ref[...], 0.0)
        h = jnp.dot(h, w2_ref[...], preferred_element_type=_f32) + b2_ref[...]
        rep_ref[...] = g_ref[0, 0] * h
        gout_ref[...] = co * h
        ci_ref[...] = ci
        cp_ref[...] = co * ci

    full = lambda s: pl.BlockSpec(s, lambda i: (0, 0))
    return pl.pallas_call(
        body,
        grid=(N // _MLP_R,),
        in_specs=[
            pl.BlockSpec((_MLP_R, D), lambda i: (i, 0)),
            pl.BlockSpec((_MLP_R, NW), lambda i: (i, 0)),
            pl.BlockSpec((_MLP_R, NW), lambda i: (i, 0)),
            full((D, D)), full((1, D)), full((D, D)), full((1, D)),
            full((1, L + 1)),
        ],
        out_specs=(
            pl.BlockSpec((_MLP_R, D), lambda i: (i, 0)),
            pl.BlockSpec((_MLP_R, D), lambda i: (i, 0)),
            pl.BlockSpec((_MLP_R, 1), lambda i: (i, 0)),
            pl.BlockSpec((_MLP_R, 1), lambda i: (i, 0)),
        ),
        out_shape=(
            jax.ShapeDtypeStruct((N, D), _f32),
            jax.ShapeDtypeStruct((N, D), _f32),
            jax.ShapeDtypeStruct((N, 1), _f32),
            jax.ShapeDtypeStruct((N, 1), _f32),
        ),
    )(x, dop_t, dip_t, w1, b1, w2, b2, gam)


def _scale_tc(part, rep, cin, cprod, gam, layer, last):
    def body(part_ref, rep_ref, ci_ref, cp_ref, g_ref, *outs):
        s = part_ref[...]
        outs[0][...] = rep_ref[...] + g_ref[0, layer + 1] * (ci_ref[...] * s)
        if not last:
            outs[1][...] = cp_ref[...] * s

    n_out = 1 if last else 2
    out = pl.pallas_call(
        body,
        grid=(N // _MLP_R,),
        in_specs=[
            pl.BlockSpec((_MLP_R, D), lambda i: (i, 0)),
            pl.BlockSpec((_MLP_R, D), lambda i: (i, 0)),
            pl.BlockSpec((_MLP_R, 1), lambda i: (i, 0)),
            pl.BlockSpec((_MLP_R, 1), lambda i: (i, 0)),
            pl.BlockSpec((1, L + 1), lambda i: (0, 0)),
        ],
        out_specs=(pl.BlockSpec((_MLP_R, D), lambda i: (i, 0)),) * n_out,
        out_shape=(jax.ShapeDtypeStruct((N, D), _f32),) * n_out,
    )(part, rep, cin, cprod, gam)
    return out if not last else (out[0], None)


_PRED_R = 2048


def _pred_tc(z, w1, b1, w2, b2, w3, b3):
    def body(z_ref, w1_ref, b1_ref, w2_ref, b2_ref, w3_ref, b3_ref, o_ref):
        a = jnp.dot(z_ref[...], w1_ref[...], preferred_element_type=_f32)
        a = jnp.maximum(a + b1_ref[...], 0.0)
        a = jnp.dot(a, w2_ref[...], preferred_element_type=_f32)
        a = jnp.maximum(a + b2_ref[...], 0.0)
        o_ref[...] = (jnp.dot(a, w3_ref[...], preferred_element_type=_f32)
                      + b3_ref[...])

    full = lambda s: pl.BlockSpec(s, lambda i: (0, 0))
    return pl.pallas_call(
        body,
        grid=(P2 // _PRED_R,),
        in_specs=[
            pl.BlockSpec((_PRED_R, D), lambda i: (i, 0)),
            full((D, D)), full((1, D)), full((D, D)), full((1, D)),
            full((D, 1)), full((1, 1)),
        ],
        out_specs=pl.BlockSpec((_PRED_R, 1), lambda i: (i, 0)),
        out_shape=jax.ShapeDtypeStruct((P2, 1), _f32),
    )(z, w1, b1, w2, b2, w3, b3)


def kernel(x, edge_index, pos_edge_index, neg_edge_index, lin1_W, lin1_b,
           lin2_W, lin2_b, gamma, p1_W, p1_b, p2_W, p2_b, p3_W, p3_b):
    gam = gamma.reshape(1, L + 1)
    _pre_sc, _conv_sc, _pair_sc = _sc_kernels()

    dop, dip, lsrc, ldst, cnt = _pre_sc(edge_index[0].reshape(NW, DECH, EBW),
                                        edge_index[1].reshape(NW, DECH, EBW))
    lsrc_r = lsrc.reshape(NC, NW, LCH, ECW)
    ldst_r = ldst.reshape(NC, NW, LCH, ECW)
    rep, g, cin, cprod = _mlp_tc(x, dop.T, dip.T, lin1_W,
                                 lin1_b.reshape(1, D), lin2_W,
                                 lin2_b.reshape(1, D), gam)

    zer = jnp.zeros((HROW, D), _f32)
    for l in range(L):
        part = _conv_sc(g, lsrc_r, ldst_r, cnt, zer).reshape(NPAD, D)[:N]
        rep, g = _scale_tc(part, rep, cin, cprod, gam, l, last=(l == L - 1))

    pad = jnp.zeros((P2 - 2 * EP,), _i32)
    ia = jnp.concatenate([pos_edge_index[0], neg_edge_index[0], pad])
    ib = jnp.concatenate([pos_edge_index[1], neg_edge_index[1], pad])
    z = _pair_sc(rep, ia.reshape(NW, PCH, PCW), ib.reshape(NW, PCH, PCW))

    scores = _pred_tc(z, p1_W, p1_b.reshape(1, D), p2_W, p2_b.reshape(1, D),
                      p3_W, p3_b.reshape(1, 1))
    return scores[:EP], scores[EP:2 * EP]
